# baseline (speedup 1.0000x reference)
import functools

import jax
import jax.numpy as jnp
from jax import lax
from jax.experimental import pallas as pl
from jax.experimental.pallas import tpu as pltpu
from jax.experimental.pallas import tpu_sc as plsc

N_SPEC = 512
N_REACT = 32768
N_RATES = 16384
BATCH = 512

NUM_CORES = 2
NUM_SUBCORES = 16
NW = NUM_CORES * NUM_SUBCORES
ROWS_PER_W = BATCH // NW
LANES = 16
ROW_BLK = 2
N_BLKS = ROWS_PER_W // ROW_BLK


def _sc_body(y_hbm, rates_hbm, den_hbm, pa_hbm, pb_hbm, out_hbm,
             pa_v, pb_v, y0_v, y1_v, r0_v, r1_v, acc0_v, acc1_v, den_v,
             dma_sem):
    wid = lax.axis_index("s") * NUM_CORES + lax.axis_index("c")
    row0 = wid * ROWS_PER_W
    pltpu.sync_copy(pa_hbm, pa_v)
    pltpu.sync_copy(pb_hbm, pb_v)
    pltpu.sync_copy(den_hbm.at[pl.ds(row0, ROWS_PER_W)], den_v)

    def blk_body(rb, carry):
        row = row0 + rb * ROW_BLK
        copies = [pltpu.async_copy(y_hbm.at[row], y0_v, dma_sem),
                  pltpu.async_copy(y_hbm.at[row + 1], y1_v, dma_sem),
                  pltpu.async_copy(rates_hbm.at[row], r0_v, dma_sem),
                  pltpu.async_copy(rates_hbm.at[row + 1], r1_v, dma_sem)]

        def zero_body(i, c):
            acc0_v[pl.ds(i * LANES, LANES)] = jnp.zeros((LANES,), jnp.float32)
            acc1_v[pl.ds(i * LANES, LANES)] = jnp.zeros((LANES,), jnp.float32)
            return c
        lax.fori_loop(0, (2 * N_SPEC) // LANES, zero_body, 0)
        for h in copies:
            h.wait()

        @plsc.parallel_loop(0, N_REACT, step=LANES, unroll=16)
        def grp_body(i):
            pa = pa_v[pl.ds(i, LANES)]
            pb = pb_v[pl.ds(i, LANES)]
            i0 = pa & 511
            i1 = pa >> 9
            kk = pb & 16383
            px = pb >> 14
            for y_v, r_v, acc_v in ((y0_v, r0_v, acc0_v),
                                    (y1_v, r1_v, acc1_v)):
                ya = plsc.load_gather(y_v, [i0])
                yb = plsc.load_gather(y_v, [i1])
                rr = plsc.load_gather(r_v, [kk])
                plsc.addupdate_scatter(acc_v, [px], ya * yb * rr)

        for g, acc_v in ((0, acc0_v), (1, acc1_v)):
            denv = plsc.load_gather(
                den_v, [jnp.full((LANES,), rb * ROW_BLK + g, jnp.int32)])

            def fin_body(i, c, acc_v=acc_v, denv=denv):
                sl = pl.ds(i * LANES, LANES)
                neg = acc_v[pl.ds(i * LANES + N_SPEC, LANES)]
                acc_v[sl] = (acc_v[sl] - neg) * denv
                return c
            lax.fori_loop(0, N_SPEC // LANES, fin_body, 0)
            pltpu.sync_copy(acc_v.at[pl.ds(0, N_SPEC)], out_hbm.at[row + g])
        return carry

    lax.fori_loop(0, N_BLKS, blk_body, 0)


_sc_kernel = functools.partial(
    pl.kernel,
    out_type=jax.ShapeDtypeStruct((BATCH, N_SPEC), jnp.float32),
    mesh=plsc.VectorSubcoreMesh(core_axis_name="c", subcore_axis_name="s",
                                num_cores=NUM_CORES,
                                num_subcores=NUM_SUBCORES),
    compiler_params=pltpu.CompilerParams(needs_layout_passes=False),
    scratch_types=[
        pltpu.VMEM((N_REACT,), jnp.int32),
        pltpu.VMEM((N_REACT,), jnp.int32),
        pltpu.VMEM((N_SPEC,), jnp.float32),
        pltpu.VMEM((N_SPEC,), jnp.float32),
        pltpu.VMEM((N_RATES,), jnp.float32),
        pltpu.VMEM((N_RATES,), jnp.float32),
        pltpu.VMEM((2 * N_SPEC,), jnp.float32),
        pltpu.VMEM((2 * N_SPEC,), jnp.float32),
        pltpu.VMEM((ROWS_PER_W,), jnp.float32),
        pltpu.SemaphoreType.DMA,
    ],
)(_sc_body)


def kernel(y_in, rates, den_norm, inds_r, inds_p, inds_k, rate_sign):
    ir0 = inds_r[:, 0].astype(jnp.int32)
    ir1 = inds_r[:, 1].astype(jnp.int32)
    ik = inds_k.astype(jnp.int32)
    ipx = inds_p.astype(jnp.int32) + jnp.where(rate_sign < 0, N_SPEC, 0)
    packed_a = ir0 | (ir1 << 9)
    packed_b = ik | (ipx << 14)
    den = den_norm.reshape(-1).astype(jnp.float32)
    return _sc_kernel(y_in, rates, den, packed_a, packed_b)


# speedup vs baseline: 1.1435x; 1.1435x over previous
import functools

import jax
import jax.numpy as jnp
from jax import lax
from jax.experimental import pallas as pl
from jax.experimental.pallas import tpu as pltpu
from jax.experimental.pallas import tpu_sc as plsc

N_SPEC = 512
N_REACT = 32768
N_RATES = 16384
BATCH = 512

NUM_CORES = 2
NUM_SUBCORES = 16
NW = NUM_CORES * NUM_SUBCORES
ROWS_PER_W = BATCH // NW
LANES = 16
ROW_BLK = 4
N_BLKS = ROWS_PER_W // ROW_BLK
ICHUNK = 4096
N_ICHUNKS = N_REACT // ICHUNK


def _sc_body(y_hbm, rates_hbm, den_hbm, pa_hbm, pb_hbm, out_hbm,
             pa0_v, pb0_v, pa1_v, pb1_v,
             y0_v, y1_v, y2_v, y3_v,
             r0_v, r1_v, r2_v, r3_v,
             acc0_v, acc1_v, acc2_v, acc3_v,
             den_v, row_sem, idx_sem):
    wid = lax.axis_index("s") * NUM_CORES + lax.axis_index("c")
    row0 = wid * ROWS_PER_W
    pltpu.sync_copy(den_hbm.at[pl.ds(row0, ROWS_PER_W)], den_v)

    ys = (y0_v, y1_v, y2_v, y3_v)
    rs = (r0_v, r1_v, r2_v, r3_v)
    accs = (acc0_v, acc1_v, acc2_v, acc3_v)
    idx_bufs = ((pa0_v, pb0_v), (pa1_v, pb1_v))

    def blk_body(rb, carry):
        row = row0 + rb * ROW_BLK
        copies = []
        for g in range(ROW_BLK):
            copies.append(pltpu.async_copy(y_hbm.at[row + g], ys[g], row_sem))
            copies.append(
                pltpu.async_copy(rates_hbm.at[row + g], rs[g], row_sem))

        def zero_body(i, c):
            for acc_v in accs:
                acc_v[pl.ds(i * LANES, LANES)] = jnp.zeros((LANES,),
                                                           jnp.float32)
            return c
        lax.fori_loop(0, (2 * N_SPEC) // LANES, zero_body, 0)
        for h in copies:
            h.wait()

        pend = [pltpu.async_copy(pa_hbm.at[pl.ds(0, ICHUNK)],
                                 idx_bufs[0][0], idx_sem),
                pltpu.async_copy(pb_hbm.at[pl.ds(0, ICHUNK)],
                                 idx_bufs[0][1], idx_sem)]
        for c in range(N_ICHUNKS):
            pa_v, pb_v = idx_bufs[c % 2]
            for h in pend:
                h.wait()
            if c + 1 < N_ICHUNKS:
                npa, npb = idx_bufs[(c + 1) % 2]
                off = (c + 1) * ICHUNK
                pend = [pltpu.async_copy(pa_hbm.at[pl.ds(off, ICHUNK)],
                                         npa, idx_sem),
                        pltpu.async_copy(pb_hbm.at[pl.ds(off, ICHUNK)],
                                         npb, idx_sem)]

            @plsc.parallel_loop(0, ICHUNK, step=LANES, unroll=8)
            def grp_body(i):
                pa = pa_v[pl.ds(i, LANES)]
                pb = pb_v[pl.ds(i, LANES)]
                i0 = pa & 511
                i1 = pa >> 9
                kk = pb & 16383
                px = pb >> 14
                for y_v, r_v, acc_v in zip(ys, rs, accs):
                    ya = plsc.load_gather(y_v, [i0])
                    yb = plsc.load_gather(y_v, [i1])
                    rr = plsc.load_gather(r_v, [kk])
                    plsc.addupdate_scatter(acc_v, [px], ya * yb * rr)

        for g in range(ROW_BLK):
            acc_v = accs[g]
            denv = plsc.load_gather(
                den_v, [jnp.full((LANES,), rb * ROW_BLK + g, jnp.int32)])

            def fin_body(i, c, acc_v=acc_v, denv=denv):
                sl = pl.ds(i * LANES, LANES)
                neg = acc_v[pl.ds(i * LANES + N_SPEC, LANES)]
                acc_v[sl] = (acc_v[sl] - neg) * denv
                return c
            lax.fori_loop(0, N_SPEC // LANES, fin_body, 0)
            pltpu.sync_copy(acc_v.at[pl.ds(0, N_SPEC)], out_hbm.at[row + g])
        return carry

    lax.fori_loop(0, N_BLKS, blk_body, 0)


_sc_kernel = functools.partial(
    pl.kernel,
    out_type=jax.ShapeDtypeStruct((BATCH, N_SPEC), jnp.float32),
    mesh=plsc.VectorSubcoreMesh(core_axis_name="c", subcore_axis_name="s",
                                num_cores=NUM_CORES,
                                num_subcores=NUM_SUBCORES),
    compiler_params=pltpu.CompilerParams(needs_layout_passes=False),
    scratch_types=[
        pltpu.VMEM((ICHUNK,), jnp.int32),
        pltpu.VMEM((ICHUNK,), jnp.int32),
        pltpu.VMEM((ICHUNK,), jnp.int32),
        pltpu.VMEM((ICHUNK,), jnp.int32),
        pltpu.VMEM((N_SPEC,), jnp.float32),
        pltpu.VMEM((N_SPEC,), jnp.float32),
        pltpu.VMEM((N_SPEC,), jnp.float32),
        pltpu.VMEM((N_SPEC,), jnp.float32),
        pltpu.VMEM((N_RATES,), jnp.float32),
        pltpu.VMEM((N_RATES,), jnp.float32),
        pltpu.VMEM((N_RATES,), jnp.float32),
        pltpu.VMEM((N_RATES,), jnp.float32),
        pltpu.VMEM((2 * N_SPEC,), jnp.float32),
        pltpu.VMEM((2 * N_SPEC,), jnp.float32),
        pltpu.VMEM((2 * N_SPEC,), jnp.float32),
        pltpu.VMEM((2 * N_SPEC,), jnp.float32),
        pltpu.VMEM((ROWS_PER_W,), jnp.float32),
        pltpu.SemaphoreType.DMA,
        pltpu.SemaphoreType.DMA,
    ],
)(_sc_body)


def kernel(y_in, rates, den_norm, inds_r, inds_p, inds_k, rate_sign):
    ir0 = inds_r[:, 0].astype(jnp.int32)
    ir1 = inds_r[:, 1].astype(jnp.int32)
    ik = inds_k.astype(jnp.int32)
    ipx = inds_p.astype(jnp.int32) + jnp.where(rate_sign < 0, N_SPEC, 0)
    packed_a = ir0 | (ir1 << 9)
    packed_b = ik | (ipx << 14)
    den = den_norm.reshape(-1).astype(jnp.float32)
    return _sc_kernel(y_in, rates, den, packed_a, packed_b)


# speedup vs baseline: 1.2079x; 1.0563x over previous
import functools

import jax
import jax.numpy as jnp
from jax import lax
from jax.experimental import pallas as pl
from jax.experimental.pallas import tpu as pltpu
from jax.experimental.pallas import tpu_sc as plsc

N_SPEC = 512
N_REACT = 32768
N_RATES = 16384
BATCH = 512

NUM_CORES = 2
NUM_SUBCORES = 16
NW = NUM_CORES * NUM_SUBCORES
ROWS_PER_W = BATCH // NW
LANES = 16
ROW_BLK = 4
N_BLKS = ROWS_PER_W // ROW_BLK
ICHUNK = 4096
N_ICHUNKS = N_REACT // ICHUNK
ACC_WORDS = N_SPEC * LANES


def _sc_body(y_hbm, rates_hbm, den_hbm, pa_hbm, pb_hbm, out_hbm,
             pa0_v, pb0_v, pa1_v, pb1_v,
             y0_v, y1_v, y2_v, y3_v,
             r0_v, r1_v, r2_v, r3_v,
             acc0_v, acc1_v, acc2_v, acc3_v,
             den_v, row_sem, idx_sem):
    wid = lax.axis_index("s") * NUM_CORES + lax.axis_index("c")
    row0 = wid * ROWS_PER_W
    pltpu.sync_copy(den_hbm.at[pl.ds(row0, ROWS_PER_W)], den_v)

    ys = (y0_v, y1_v, y2_v, y3_v)
    rs = (r0_v, r1_v, r2_v, r3_v)
    accs = (acc0_v, acc1_v, acc2_v, acc3_v)
    idx_bufs = ((pa0_v, pb0_v), (pa1_v, pb1_v))

    def init_body(i, c):
        for acc_v in accs:
            acc_v[pl.ds(i * LANES, LANES)] = jnp.zeros((LANES,), jnp.float32)
        return c
    lax.fori_loop(0, ACC_WORDS // LANES, init_body, 0)

    def blk_body(rb, carry):
        row = row0 + rb * ROW_BLK
        copies = []
        for g in range(ROW_BLK):
            copies.append(pltpu.async_copy(y_hbm.at[row + g], ys[g], row_sem))
            copies.append(
                pltpu.async_copy(rates_hbm.at[row + g], rs[g], row_sem))

        pend = [pltpu.async_copy(pa_hbm.at[pl.ds(0, ICHUNK)],
                                 idx_bufs[0][0], idx_sem),
                pltpu.async_copy(pb_hbm.at[pl.ds(0, ICHUNK)],
                                 idx_bufs[0][1], idx_sem)]
        for h in copies:
            h.wait()
        for c in range(N_ICHUNKS):
            pa_v, pb_v = idx_bufs[c % 2]
            for h in pend:
                h.wait()
            if c + 1 < N_ICHUNKS:
                npa, npb = idx_bufs[(c + 1) % 2]
                off = (c + 1) * ICHUNK
                pend = [pltpu.async_copy(pa_hbm.at[pl.ds(off, ICHUNK)],
                                         npa, idx_sem),
                        pltpu.async_copy(pb_hbm.at[pl.ds(off, ICHUNK)],
                                         npb, idx_sem)]

            @plsc.parallel_loop(0, ICHUNK, step=LANES, unroll=8)
            def grp_body(i):
                lane = lax.iota(jnp.int32, LANES)
                pa = pa_v[pl.ds(i, LANES)]
                pb = pb_v[pl.ds(i, LANES)]
                i0 = pa & 511
                i1 = pa >> 9
                kk = pb & 16383
                px = ((pb >> 14) & 511) * LANES + lane
                neg = (pb >> 23) == 1
                for y_v, r_v, acc_v in zip(ys, rs, accs):
                    ya = plsc.load_gather(y_v, [i0])
                    yb = plsc.load_gather(y_v, [i1])
                    rr = plsc.load_gather(r_v, [kk])
                    t = ya * yb * rr
                    plsc.addupdate_scatter(acc_v, [px], jnp.where(neg, -t, t))

        for g in range(ROW_BLK):
            acc_v = accs[g]
            y_v = ys[g]
            denv = plsc.load_gather(
                den_v, [jnp.full((LANES,), rb * ROW_BLK + g, jnp.int32)])

            def fin_body(j, c, acc_v=acc_v, y_v=y_v, denv=denv):
                lane = lax.iota(jnp.int32, LANES)
                bvec = (j * LANES + lane) * LANES
                tot = jnp.zeros((LANES,), jnp.float32)
                for l in range(LANES):
                    idx = bvec + ((lane + l) & (LANES - 1))
                    tot = tot + plsc.load_gather(acc_v, [idx])
                    plsc.store_scatter(acc_v, [idx],
                                       jnp.zeros((LANES,), jnp.float32))
                y_v[pl.ds(j * LANES, LANES)] = tot * denv
                return c
            lax.fori_loop(0, N_SPEC // LANES, fin_body, 0)
            pltpu.sync_copy(y_v, out_hbm.at[row + g])
        return carry

    lax.fori_loop(0, N_BLKS, blk_body, 0)


_sc_kernel = functools.partial(
    pl.kernel,
    out_type=jax.ShapeDtypeStruct((BATCH, N_SPEC), jnp.float32),
    mesh=plsc.VectorSubcoreMesh(core_axis_name="c", subcore_axis_name="s",
                                num_cores=NUM_CORES,
                                num_subcores=NUM_SUBCORES),
    compiler_params=pltpu.CompilerParams(needs_layout_passes=False),
    scratch_types=[
        pltpu.VMEM((ICHUNK,), jnp.int32),
        pltpu.VMEM((ICHUNK,), jnp.int32),
        pltpu.VMEM((ICHUNK,), jnp.int32),
        pltpu.VMEM((ICHUNK,), jnp.int32),
        pltpu.VMEM((N_SPEC,), jnp.float32),
        pltpu.VMEM((N_SPEC,), jnp.float32),
        pltpu.VMEM((N_SPEC,), jnp.float32),
        pltpu.VMEM((N_SPEC,), jnp.float32),
        pltpu.VMEM((N_RATES,), jnp.float32),
        pltpu.VMEM((N_RATES,), jnp.float32),
        pltpu.VMEM((N_RATES,), jnp.float32),
        pltpu.VMEM((N_RATES,), jnp.float32),
        pltpu.VMEM((ACC_WORDS,), jnp.float32),
        pltpu.VMEM((ACC_WORDS,), jnp.float32),
        pltpu.VMEM((ACC_WORDS,), jnp.float32),
        pltpu.VMEM((ACC_WORDS,), jnp.float32),
        pltpu.VMEM((ROWS_PER_W,), jnp.float32),
        pltpu.SemaphoreType.DMA,
        pltpu.SemaphoreType.DMA,
    ],
)(_sc_body)


def kernel(y_in, rates, den_norm, inds_r, inds_p, inds_k, rate_sign):
    ir0 = inds_r[:, 0].astype(jnp.int32)
    ir1 = inds_r[:, 1].astype(jnp.int32)
    ik = inds_k.astype(jnp.int32)
    ip = inds_p.astype(jnp.int32)
    sgn = jnp.where(rate_sign < 0, 1, 0).astype(jnp.int32)
    packed_a = ir0 | (ir1 << 9)
    packed_b = ik | (ip << 14) | (sgn << 23)
    den = den_norm.reshape(-1).astype(jnp.float32)
    return _sc_kernel(y_in, rates, den, packed_a, packed_b)


# speedup vs baseline: 1.2692x; 1.0508x over previous
import functools

import jax
import jax.numpy as jnp
from jax import lax
from jax.experimental import pallas as pl
from jax.experimental.pallas import tpu as pltpu
from jax.experimental.pallas import tpu_sc as plsc

N_SPEC = 512
N_REACT = 32768
N_RATES = 16384
BATCH = 512

NUM_CORES = 2
NUM_SUBCORES = 16
NW = NUM_CORES * NUM_SUBCORES
ROWS_PER_W = BATCH // NW
LANES = 16
ROW_BLK = 4
N_BLKS = ROWS_PER_W // ROW_BLK
ICHUNK = 4096
N_ICHUNKS = N_REACT // ICHUNK
ACC_WORDS = N_SPEC * LANES


def _sc_body(y_hbm, rates_hbm, den_hbm, pa_hbm, pb_hbm, out_hbm,
             pa0_v, pb0_v, pa1_v, pb1_v,
             y0_v, y1_v, y2_v, y3_v,
             r0_v, r1_v, r2_v, r3_v,
             acc0_v, acc1_v, acc2_v, acc3_v,
             den_v, stage_v, row_sem, idx_sem):
    wid = lax.axis_index("s") * NUM_CORES + lax.axis_index("c")
    row0 = wid * ROWS_PER_W
    pltpu.sync_copy(den_hbm.at[pl.ds(row0, ROWS_PER_W)], den_v)

    ys = (y0_v, y1_v, y2_v, y3_v)
    rs = (r0_v, r1_v, r2_v, r3_v)
    accs = (acc0_v, acc1_v, acc2_v, acc3_v)
    idx_bufs = ((pa0_v, pb0_v), (pa1_v, pb1_v))

    def init_body(i, c):
        for acc_v in accs:
            acc_v[pl.ds(i * LANES, LANES)] = jnp.zeros((LANES,), jnp.float32)
        return c
    lax.fori_loop(0, ACC_WORDS // LANES, init_body, 0)

    def finalize_blk(rbm1):
        for g in range(ROW_BLK):
            acc_v = accs[g]
            denv = plsc.load_gather(
                den_v, [jnp.full((LANES,), rbm1 * ROW_BLK + g, jnp.int32)])

            def fin_body(j, c, acc_v=acc_v, denv=denv):
                lane = lax.iota(jnp.int32, LANES)
                bvec = (j * LANES + lane) * LANES
                tot = jnp.zeros((LANES,), jnp.float32)
                for l in range(LANES):
                    idx = bvec + ((lane + l) & (LANES - 1))
                    tot = tot + plsc.load_gather(acc_v, [idx])
                    plsc.store_scatter(acc_v, [idx],
                                       jnp.zeros((LANES,), jnp.float32))
                stage_v[pl.ds(j * LANES, LANES)] = tot * denv
                return c
            lax.fori_loop(0, N_SPEC // LANES, fin_body, 0)
            pltpu.sync_copy(stage_v, out_hbm.at[row0 + rbm1 * ROW_BLK + g])

    def blk_body(rb, carry):
        row = row0 + rb * ROW_BLK
        copies = []
        for g in range(ROW_BLK):
            copies.append(pltpu.async_copy(y_hbm.at[row + g], ys[g], row_sem))
            copies.append(
                pltpu.async_copy(rates_hbm.at[row + g], rs[g], row_sem))

        pend = [pltpu.async_copy(pa_hbm.at[pl.ds(0, ICHUNK)],
                                 idx_bufs[0][0], idx_sem),
                pltpu.async_copy(pb_hbm.at[pl.ds(0, ICHUNK)],
                                 idx_bufs[0][1], idx_sem)]

        @pl.when(rb > 0)
        def _():
            finalize_blk(rb - 1)

        for h in copies:
            h.wait()
        for c in range(N_ICHUNKS):
            pa_v, pb_v = idx_bufs[c % 2]
            for h in pend:
                h.wait()
            if c + 1 < N_ICHUNKS:
                npa, npb = idx_bufs[(c + 1) % 2]
                off = (c + 1) * ICHUNK
                pend = [pltpu.async_copy(pa_hbm.at[pl.ds(off, ICHUNK)],
                                         npa, idx_sem),
                        pltpu.async_copy(pb_hbm.at[pl.ds(off, ICHUNK)],
                                         npb, idx_sem)]

            @plsc.parallel_loop(0, ICHUNK, step=LANES, unroll=8)
            def grp_body(i):
                lane = lax.iota(jnp.int32, LANES)
                pa = pa_v[pl.ds(i, LANES)]
                pb = pb_v[pl.ds(i, LANES)]
                i0 = pa & 511
                i1 = pa >> 9
                kk = pb & 16383
                px = ((pb >> 14) & 511) * LANES + lane
                neg = (pb >> 23) == 1
                for y_v, r_v, acc_v in zip(ys, rs, accs):
                    ya = plsc.load_gather(y_v, [i0])
                    yb = plsc.load_gather(y_v, [i1])
                    rr = plsc.load_gather(r_v, [kk])
                    t = ya * yb * rr
                    plsc.addupdate_scatter(acc_v, [px], jnp.where(neg, -t, t))

        return carry

    lax.fori_loop(0, N_BLKS, blk_body, 0)
    finalize_blk(N_BLKS - 1)


_sc_kernel = functools.partial(
    pl.kernel,
    out_type=jax.ShapeDtypeStruct((BATCH, N_SPEC), jnp.float32),
    mesh=plsc.VectorSubcoreMesh(core_axis_name="c", subcore_axis_name="s",
                                num_cores=NUM_CORES,
                                num_subcores=NUM_SUBCORES),
    compiler_params=pltpu.CompilerParams(needs_layout_passes=False),
    scratch_types=[
        pltpu.VMEM((ICHUNK,), jnp.int32),
        pltpu.VMEM((ICHUNK,), jnp.int32),
        pltpu.VMEM((ICHUNK,), jnp.int32),
        pltpu.VMEM((ICHUNK,), jnp.int32),
        pltpu.VMEM((N_SPEC,), jnp.float32),
        pltpu.VMEM((N_SPEC,), jnp.float32),
        pltpu.VMEM((N_SPEC,), jnp.float32),
        pltpu.VMEM((N_SPEC,), jnp.float32),
        pltpu.VMEM((N_RATES,), jnp.float32),
        pltpu.VMEM((N_RATES,), jnp.float32),
        pltpu.VMEM((N_RATES,), jnp.float32),
        pltpu.VMEM((N_RATES,), jnp.float32),
        pltpu.VMEM((ACC_WORDS,), jnp.float32),
        pltpu.VMEM((ACC_WORDS,), jnp.float32),
        pltpu.VMEM((ACC_WORDS,), jnp.float32),
        pltpu.VMEM((ACC_WORDS,), jnp.float32),
        pltpu.VMEM((ROWS_PER_W,), jnp.float32),
        pltpu.VMEM((N_SPEC,), jnp.float32),
        pltpu.SemaphoreType.DMA,
        pltpu.SemaphoreType.DMA,
    ],
)(_sc_body)


def kernel(y_in, rates, den_norm, inds_r, inds_p, inds_k, rate_sign):
    ir0 = inds_r[:, 0].astype(jnp.int32)
    ir1 = inds_r[:, 1].astype(jnp.int32)
    ik = inds_k.astype(jnp.int32)
    ip = inds_p.astype(jnp.int32)
    sgn = jnp.where(rate_sign < 0, 1, 0).astype(jnp.int32)
    packed_a = ir0 | (ir1 << 9)
    packed_b = ik | (ip << 14) | (sgn << 23)
    den = den_norm.reshape(-1).astype(jnp.float32)
    return _sc_kernel(y_in, rates, den, packed_a, packed_b)
